# baseline (speedup 1.0000x reference)
import functools

import jax
import jax.numpy as jnp
from jax import lax
from jax.experimental import pallas as pl
from jax.experimental.pallas import tpu as pltpu
from jax.experimental.pallas import tpu_sc as plsc

DIM = 32
_INFO = plsc.get_sparse_core_info()
NC = _INFO.num_cores
NS = _INFO.num_subcores
NW = NC * NS

CHUNK = 128


def _make_gather(n_total: int):
  assert n_total % (NW * CHUNK) == 0
  bpw = n_total // NW
  nch = bpw // CHUNK
  mesh = plsc.VectorSubcoreMesh(core_axis_name="c", subcore_axis_name="s")

  @functools.partial(
      pl.kernel,
      mesh=mesh,
      compiler_params=pltpu.CompilerParams(use_tc_tiling_on_sc=False),
      out_type=jax.ShapeDtypeStruct((n_total, DIM), jnp.float32),
      scratch_types=[
          pltpu.VMEM((nch, CHUNK), jnp.int32),
          pltpu.VMEM((CHUNK, DIM), jnp.float32),
          pltpu.SemaphoreType.DMA,
      ],
  )
  def gather_kernel(keys_hbm, table_hbm, out_hbm, idx_v, rows_v, sem):
    wid = lax.axis_index("s") * NC + lax.axis_index("c")
    base = wid * bpw
    pltpu.sync_copy(keys_hbm.at[wid], idx_v)

    @pl.loop(0, nch)
    def _body(c):
      pltpu.async_copy(table_hbm.at[idx_v.at[c]], rows_v, sem).wait()
      pltpu.sync_copy(rows_v, out_hbm.at[pl.ds(base + c * CHUNK, CHUNK)])

  return gather_kernel


@jax.jit
def kernel(keys, table):
  b, l = keys.shape
  n_total = b * l
  keys_flat = keys.reshape(NW, n_total // (NW * CHUNK), CHUNK)
  out = _make_gather(n_total)(keys_flat, table)
  return out.reshape(b, l, DIM)


# speedup vs baseline: 1.2455x; 1.2455x over previous
import functools

import jax
import jax.numpy as jnp
from jax import lax
from jax.experimental import pallas as pl
from jax.experimental.pallas import tpu as pltpu
from jax.experimental.pallas import tpu_sc as plsc

DIM = 32
_INFO = plsc.get_sparse_core_info()
NC = _INFO.num_cores
NS = _INFO.num_subcores
NW = NC * NS

CHUNK = 128


def _make_gather(n_total: int, vocab: int):
  assert n_total % (NW * CHUNK) == 0
  bpw = n_total // NW
  nch = bpw // CHUNK
  assert nch % 2 == 0
  mesh = plsc.VectorSubcoreMesh(core_axis_name="c", subcore_axis_name="s")

  @functools.partial(
      pl.kernel,
      mesh=mesh,
      compiler_params=pltpu.CompilerParams(use_tc_tiling_on_sc=False),
      out_type=jax.ShapeDtypeStruct((n_total, DIM), jnp.float32),
      scratch_types=[
          pltpu.VMEM((bpw,), jnp.int32),
          pltpu.VMEM((CHUNK, DIM), jnp.float32),
          pltpu.VMEM((CHUNK, DIM), jnp.float32),
          pltpu.SemaphoreType.DMA,
          pltpu.SemaphoreType.DMA,
      ],
  )
  def gather_kernel(keys_hbm, table_hbm, out_hbm, idx_v, rows_a, rows_b,
                    sem_a, sem_b):
    wid = lax.axis_index("s") * NC + lax.axis_index("c")
    pltpu.sync_copy(keys_hbm.at[pl.ds(wid * bpw, bpw)], idx_v)
    obase = wid * bpw

    def gather(c, buf, sem):
      return pltpu.async_copy(
          table_hbm.at[idx_v.at[pl.ds(c * CHUNK, CHUNK)]], buf, sem)

    def gather_wait(c, buf, sem):
      pltpu.make_async_copy(
          table_hbm.at[idx_v.at[pl.ds(c * CHUNK, CHUNK)]], buf, sem).wait()

    gather(0, rows_a, sem_a)

    @pl.loop(0, nch, step=2)
    def _body(c):
      gather(c + 1, rows_b, sem_b)
      gather_wait(c, rows_a, sem_a)
      pltpu.sync_copy(rows_a, out_hbm.at[pl.ds(obase + c * CHUNK, CHUNK)])

      @pl.when(c + 2 < nch)
      def _():
        gather(c + 2, rows_a, sem_a)

      gather_wait(c + 1, rows_b, sem_b)
      pltpu.sync_copy(rows_b,
                      out_hbm.at[pl.ds(obase + (c + 1) * CHUNK, CHUNK)])

  return gather_kernel


@jax.jit
def kernel(keys, table):
  b, l = keys.shape
  vocab, dim = table.shape
  n_total = b * l
  keys_m = keys.T.reshape(l, NW, CHUNK).transpose(1, 0, 2).reshape(-1)
  out = _make_gather(n_total, vocab)(keys_m, table)
  out = out.reshape(NW, l, CHUNK, dim).transpose(0, 2, 1, 3)
  return out.reshape(b, l, dim)
